# speedup vs baseline: 1.9104x; 1.9104x over previous
import jax
import jax.numpy as jnp
from jax.experimental import pallas as pl


def _support1_body(feat_ref, feat_a_ref, W_ref, s1_ref):
    W = W_ref[...]
    s1_ref[:, : W.shape[1]] = jnp.dot(
        feat_ref[...], W, preferred_element_type=jnp.float32
    )
    s1_ref[:, W.shape[1] :] = jnp.dot(
        feat_a_ref[...], W, preferred_element_type=jnp.float32
    )


def _layer1_body(adj_ref, s1_ref, W2_ref, z_ori_ref, z_aug_ref, s2_ref):
    z = jnp.dot(adj_ref[...], s1_ref[...], preferred_element_type=jnp.float32)
    h = z.shape[1] // 2
    z_ori_ref[...] = z[:, :h]
    z_aug_ref[...] = z[:, h:]
    s2_ref[...] = jnp.dot(z, W2_ref[...], preferred_element_type=jnp.float32)


def _layer2_body(adj_ref, s2_ref, x_ori_ref, x_aug_ref):
    x = jnp.dot(adj_ref[...], s2_ref[...], preferred_element_type=jnp.float32)
    h = x.shape[1] // 2
    x_ori_ref[...] = x[:, :h]
    x_aug_ref[...] = x[:, h:]


@jax.jit
def kernel(feat, feat_a, fadj, W_z, W_x):
    n, _ = feat.shape
    nhid = W_z.shape[1]
    nout = W_x.shape[1]
    f32 = jnp.float32

    bm = 400
    if n % bm != 0:
        for cand in (200, 100, 50, 25, 8, 5, 4, 2, 1):
            if n % cand == 0:
                bm = cand
                break
    grid = (n // bm,)

    s1 = pl.pallas_call(
        _support1_body,
        out_shape=jax.ShapeDtypeStruct((n, 2 * nhid), f32),
    )(feat, feat_a, W_z)

    zeros = jnp.zeros((nhid, nout), f32)
    W2 = jnp.block([[W_x, zeros], [zeros, W_x]])

    z_ori, z_aug, s2 = pl.pallas_call(
        _layer1_body,
        grid=grid,
        in_specs=[
            pl.BlockSpec((bm, n), lambda i: (i, 0)),
            pl.BlockSpec((n, 2 * nhid), lambda i: (0, 0)),
            pl.BlockSpec((2 * nhid, 2 * nout), lambda i: (0, 0)),
        ],
        out_specs=[
            pl.BlockSpec((bm, nhid), lambda i: (i, 0)),
            pl.BlockSpec((bm, nhid), lambda i: (i, 0)),
            pl.BlockSpec((bm, 2 * nout), lambda i: (i, 0)),
        ],
        out_shape=[
            jax.ShapeDtypeStruct((n, nhid), f32),
            jax.ShapeDtypeStruct((n, nhid), f32),
            jax.ShapeDtypeStruct((n, 2 * nout), f32),
        ],
    )(fadj, s1, W2)

    xhat_ori, xhat_aug = pl.pallas_call(
        _layer2_body,
        grid=grid,
        in_specs=[
            pl.BlockSpec((bm, n), lambda i: (i, 0)),
            pl.BlockSpec((n, 2 * nout), lambda i: (0, 0)),
        ],
        out_specs=[
            pl.BlockSpec((bm, nout), lambda i: (i, 0)),
            pl.BlockSpec((bm, nout), lambda i: (i, 0)),
        ],
        out_shape=[
            jax.ShapeDtypeStruct((n, nout), f32),
            jax.ShapeDtypeStruct((n, nout), f32),
        ],
    )(fadj, s2)

    return (z_ori, z_aug, xhat_ori, xhat_aug)
